# baseline (speedup 1.0000x reference)
import functools

import jax
import jax.numpy as jnp
from jax import lax
from jax.experimental import pallas as pl
from jax.experimental.pallas import tpu as pltpu
from jax.experimental.pallas import tpu_sc as plsc

L = 16384
DIM = 64
VOCAB = 1000000
HIST = 1 << 20
NC = 2
NS = 16
NW = NC * NS
PER_TILE = L // NW
SCHUNK = 128
NSCHUNK = PER_TILE // SCHUNK
ZBUF = 8192
ZREP = HIST // NS // ZBUF

BLK = 65536
NBLK = -(-VOCAB // BLK)


def _sc_histograms(token_ids):
    mesh = plsc.VectorSubcoreMesh(core_axis_name="c", subcore_axis_name="s")

    @functools.partial(
        pl.kernel,
        mesh=mesh,
        out_type=(jax.ShapeDtypeStruct((HIST,), jnp.float32),
                  jax.ShapeDtypeStruct((HIST,), jnp.float32)),
        scratch_types=[
            pltpu.VMEM((NSCHUNK, SCHUNK), jnp.int32),
            pltpu.VMEM((SCHUNK,), jnp.float32),
            pltpu.VMEM((ZBUF,), jnp.float32),
            pltpu.VMEM_SHARED((HIST,), jnp.float32),
            pltpu.SemaphoreType.DMA,
        ],
    )
    def k(tok_hbm, out0_hbm, out1_hbm, idx_v, ones_v, zbuf_v, hist_sh, zsem):
        c = lax.axis_index("c")
        s = lax.axis_index("s")
        wid = s * NC + c
        base = wid * PER_TILE

        for q in range(NSCHUNK):
            pltpu.sync_copy(tok_hbm.at[pl.ds(base + q * SCHUNK, SCHUNK)],
                            idx_v.at[q])

        one16 = jnp.full((16,), 1.0, jnp.float32)
        for i in range(SCHUNK // 16):
            ones_v[pl.ds(i * 16, 16)] = one16

        z16 = jnp.zeros((16,), jnp.float32)

        def zfill(i, _):
            zbuf_v[pl.ds(i * 16, 16)] = z16
            return 0

        lax.fori_loop(0, ZBUF // 16, zfill, 0)

        zdescs = [
            pltpu.async_copy(
                zbuf_v,
                hist_sh.at[pl.ds((s * ZREP + r) * ZBUF, ZBUF)],
                zsem)
            for r in range(ZREP)
        ]
        for d in zdescs:
            d.wait()

        plsc.subcore_barrier()

        for q in range(NSCHUNK):
            pltpu.sync_copy(ones_v, hist_sh.at[idx_v.at[q]], add=True)

        plsc.subcore_barrier()

        @pl.when(jnp.logical_and(s == 0, c == 0))
        def _emit0():
            pltpu.sync_copy(hist_sh, out0_hbm)

        @pl.when(jnp.logical_and(s == 0, c == 1))
        def _emit1():
            pltpu.sync_copy(hist_sh, out1_hbm)

    return k(token_ids)


def _scan_body(tab_ref, c0_ref, c1_ref, w_ref, o_ref, acc_ref):
    i = pl.program_id(0)

    @pl.when(i == 0)
    def _init():
        acc_ref[...] = jnp.zeros_like(acc_ref)

    cnt = c0_ref[...] + c1_ref[...]
    contrib = lax.dot_general(
        cnt, tab_ref[...], (((1,), (1,)), ((), ())),
        preferred_element_type=jnp.float32)
    acc_ref[...] += contrib

    @pl.when(i == NBLK - 1)
    def _fin():
        pooled = acc_ref[...] * (1.0 / L)
        o_ref[...] = lax.dot_general(
            pooled, w_ref[...], (((1,), (1,)), ((), ())),
            preferred_element_type=jnp.float32)


def kernel(token_ids, table, W):
    c0, c1 = _sc_histograms(token_ids)
    tab_t = table.T
    return pl.pallas_call(
        _scan_body,
        grid=(NBLK,),
        in_specs=[
            pl.BlockSpec((DIM, BLK), lambda i: (0, i)),
            pl.BlockSpec((1, BLK), lambda i: (0, i)),
            pl.BlockSpec((1, BLK), lambda i: (0, i)),
            pl.BlockSpec((DIM, DIM), lambda i: (0, 0)),
        ],
        out_specs=pl.BlockSpec((1, DIM), lambda i: (0, 0)),
        out_shape=jax.ShapeDtypeStruct((1, DIM), jnp.float32),
        scratch_shapes=[pltpu.VMEM((1, DIM), jnp.float32)],
    )(tab_t, c0.reshape(1, HIST), c1.reshape(1, HIST), W)


# speedup vs baseline: 1.0129x; 1.0129x over previous
import functools

import jax
import jax.numpy as jnp
from jax import lax
from jax.experimental import pallas as pl
from jax.experimental.pallas import tpu as pltpu
from jax.experimental.pallas import tpu_sc as plsc

L = 16384
DIM = 64
VOCAB = 1000000
HIST = 1 << 20
NC = 2
NS = 16
NW = NC * NS
PER_TILE = L // NW
SCHUNK = 128
NSCHUNK = PER_TILE // SCHUNK
ZBUF = 8192
ZREP = HIST // NS // ZBUF

BLK = 49152
NBLK = -(-VOCAB // BLK)


def _sc_histograms(token_ids):
    mesh = plsc.VectorSubcoreMesh(core_axis_name="c", subcore_axis_name="s")

    @functools.partial(
        pl.kernel,
        mesh=mesh,
        out_type=(jax.ShapeDtypeStruct((HIST,), jnp.float32),
                  jax.ShapeDtypeStruct((HIST,), jnp.float32)),
        scratch_types=[
            pltpu.VMEM((NSCHUNK, SCHUNK), jnp.int32),
            pltpu.VMEM((SCHUNK,), jnp.float32),
            pltpu.VMEM((ZBUF,), jnp.float32),
            pltpu.VMEM_SHARED((HIST,), jnp.float32),
            pltpu.SemaphoreType.DMA,
        ],
    )
    def k(tok_hbm, out0_hbm, out1_hbm, idx_v, ones_v, zbuf_v, hist_sh, zsem):
        c = lax.axis_index("c")
        s = lax.axis_index("s")
        wid = s * NC + c
        base = wid * PER_TILE

        for q in range(NSCHUNK):
            pltpu.sync_copy(tok_hbm.at[pl.ds(base + q * SCHUNK, SCHUNK)],
                            idx_v.at[q])

        one16 = jnp.full((16,), 1.0, jnp.float32)
        for i in range(SCHUNK // 16):
            ones_v[pl.ds(i * 16, 16)] = one16

        z16 = jnp.zeros((16,), jnp.float32)

        def zfill(i, _):
            zbuf_v[pl.ds(i * 16, 16)] = z16
            return 0

        lax.fori_loop(0, ZBUF // 16, zfill, 0)

        zdescs = [
            pltpu.async_copy(
                zbuf_v,
                hist_sh.at[pl.ds((s * ZREP + r) * ZBUF, ZBUF)],
                zsem)
            for r in range(ZREP)
        ]
        for d in zdescs:
            d.wait()

        plsc.subcore_barrier()

        for q in range(NSCHUNK):
            pltpu.sync_copy(ones_v, hist_sh.at[idx_v.at[q]], add=True)

        plsc.subcore_barrier()

        @pl.when(jnp.logical_and(s == 0, c == 0))
        def _emit0():
            pltpu.sync_copy(hist_sh, out0_hbm)

        @pl.when(jnp.logical_and(s == 0, c == 1))
        def _emit1():
            pltpu.sync_copy(hist_sh, out1_hbm)

    return k(token_ids)


def _scan_body(tab_ref, c0_ref, c1_ref, w_ref, o_ref, acc_ref):
    i = pl.program_id(0)

    @pl.when(i == 0)
    def _init():
        acc_ref[...] = jnp.zeros_like(acc_ref)

    cnt = c0_ref[...] + c1_ref[...]
    contrib = lax.dot_general(
        cnt, tab_ref[...], (((1,), (1,)), ((), ())),
        preferred_element_type=jnp.float32)
    acc_ref[...] += contrib

    @pl.when(i == NBLK - 1)
    def _fin():
        pooled = acc_ref[...] * (1.0 / L)
        o_ref[...] = lax.dot_general(
            pooled, w_ref[...], (((1,), (1,)), ((), ())),
            preferred_element_type=jnp.float32)


def kernel(token_ids, table, W):
    c0, c1 = _sc_histograms(token_ids)
    tab_t = table.T
    return pl.pallas_call(
        _scan_body,
        grid=(NBLK,),
        in_specs=[
            pl.BlockSpec((DIM, BLK), lambda i: (0, i)),
            pl.BlockSpec((1, BLK), lambda i: (0, i)),
            pl.BlockSpec((1, BLK), lambda i: (0, i)),
            pl.BlockSpec((DIM, DIM), lambda i: (0, 0)),
        ],
        out_specs=pl.BlockSpec((1, DIM), lambda i: (0, 0)),
        out_shape=jax.ShapeDtypeStruct((1, DIM), jnp.float32),
        scratch_shapes=[pltpu.VMEM((1, DIM), jnp.float32)],
    )(tab_t, c0.reshape(1, HIST), c1.reshape(1, HIST), W)
